# baseline (speedup 1.0000x reference)
import functools

import jax
import jax.numpy as jnp
from jax import lax
from jax.experimental import pallas as pl
from jax.experimental.pallas import tpu as pltpu
from jax.experimental.pallas import tpu_sc as plsc

_M, _N = 36864, 384
_TOTAL = _M * _N
_NW = 32
_PER_W = _TOTAL // _NW
_NBUF = 4
_CHUNK = 9216
_NCHUNK = _PER_W // _CHUNK
_LANES = 16

_mesh = plsc.VectorSubcoreMesh(core_axis_name="c", subcore_axis_name="s")

_scratch = (
    [pltpu.VMEM((_CHUNK,), jnp.float32) for _ in range(3 * _NBUF)]
    + [pltpu.SemaphoreType.DMA for _ in range(3 * _NBUF)]
)


@functools.partial(
    pl.kernel,
    out_type=jax.ShapeDtypeStruct((_TOTAL,), jnp.float32),
    mesh=_mesh,
    scratch_types=_scratch,
)
def _sc_add(l_hbm, r_hbm, o_hbm, *refs):
    lbuf = refs[0:_NBUF]
    rbuf = refs[_NBUF:2 * _NBUF]
    obuf = refs[2 * _NBUF:3 * _NBUF]
    sems = refs[3 * _NBUF:]
    lsem = sems[0:_NBUF]
    rsem = sems[_NBUF:2 * _NBUF]
    osem = sems[2 * _NBUF:3 * _NBUF]

    wid = lax.axis_index("s") * 2 + lax.axis_index("c")
    base = wid * _PER_W

    def hslice(ci):
        return pl.ds(base + ci * _CHUNK, _CHUNK)

    for p in range(_NBUF - 1):
        pltpu.async_copy(l_hbm.at[hslice(p)], lbuf[p], lsem[p])
        pltpu.async_copy(r_hbm.at[hslice(p)], rbuf[p], rsem[p])

    @pl.loop(0, _NCHUNK, step=_NBUF)
    def chunk_group(ci0):
        for b in range(_NBUF):
            ci = ci0 + b
            pb = (b + _NBUF - 1) % _NBUF

            @pl.when(ci + _NBUF - 1 < _NCHUNK)
            def _start_ahead():
                sl = hslice(ci + _NBUF - 1)
                pltpu.async_copy(l_hbm.at[sl], lbuf[pb], lsem[pb])
                pltpu.async_copy(r_hbm.at[sl], rbuf[pb], rsem[pb])

            pltpu.make_async_copy(l_hbm.at[hslice(ci)], lbuf[b], lsem[b]).wait()
            pltpu.make_async_copy(r_hbm.at[hslice(ci)], rbuf[b], rsem[b]).wait()

            @pl.when(ci >= _NBUF)
            def _drain_prev_out():
                pltpu.make_async_copy(
                    obuf[b], o_hbm.at[hslice(ci)], osem[b]).wait()

            lb, rb_, ob = lbuf[b], rbuf[b], obuf[b]

            def vbody(i):
                sl = pl.ds(i * _LANES, _LANES)
                ob[sl] = lb[sl] + rb_[sl]

            plsc.parallel_loop(0, _CHUNK // _LANES, 1, unroll=8)(vbody)

            pltpu.async_copy(obuf[b], o_hbm.at[hslice(ci)], osem[b])

    for b in range(_NBUF):
        pltpu.make_async_copy(obuf[b], o_hbm.at[hslice(b)], osem[b]).wait()


def kernel(left, right):
    out = _sc_add(left.reshape(_TOTAL), right.reshape(_TOTAL))
    return out.reshape(_M, _N)
